# baseline (speedup 1.0000x reference)
import jax
import jax.numpy as jnp
from jax.experimental import pallas as pl

_ROWS = 100000
_DIM = 128
_BLOCK_ROWS = 2000


def _copy_body(in_ref, out_ref):
    out_ref[...] = in_ref[...]


def kernel(weight):
    grid = _ROWS // _BLOCK_ROWS
    return pl.pallas_call(
        _copy_body,
        grid=(grid,),
        in_specs=[pl.BlockSpec((_BLOCK_ROWS, _DIM), lambda i: (i, 0))],
        out_specs=pl.BlockSpec((_BLOCK_ROWS, _DIM), lambda i: (i, 0)),
        out_shape=jax.ShapeDtypeStruct((_ROWS, _DIM), jnp.float32),
    )(weight)
